# Initial kernel scaffold; baseline (speedup 1.0000x reference)
import functools

import jax
import jax.numpy as jnp
from jax import lax
from jax.experimental import pallas as pl
from jax.experimental.pallas import tpu as pltpu
from jax.experimental.pallas import tpu_sc as plsc

N = 10000
E = 320000
B = 16
NF = 128
EF = 16
UF = 32

NC, NS = 2, 16
NW = NC * NS
NPAD = 10240
RPS = NPAD // NS
GW = 112
HW = 80
CH = 512
KI = CH // 128
NCHUNK = E // CH
EB = 2000
f32 = jnp.float32


def _pre_body(x_ref, u_ref, b_ref, We1_ref, be1_ref, Wn1x_ref, bn1_ref,
              trow_ref, tcol_ref):
    x = x_ref[...]
    We1 = We1_ref[...]
    onehot = (b_ref[...] == lax.broadcasted_iota(jnp.int32, (N, B), 1)
              ).astype(f32)
    uW = jnp.dot(u_ref[...], We1[2 * NF + EF:2 * NF + EF + UF, :],
                 preferred_element_type=f32)
    ta = (jnp.dot(x, We1[0:NF], preferred_element_type=f32)
          + jnp.dot(onehot, uW, preferred_element_type=f32) + be1_ref[...])
    tb = jnp.dot(x, Wn1x_ref[...], preferred_element_type=f32) + bn1_ref[...]
    bf = b_ref[...].astype(f32)
    trow_ref[...] = jnp.concatenate(
        [ta, tb, bf, jnp.zeros((N, GW - 97), f32)], axis=1)
    tcol_ref[...] = jnp.dot(x, We1[NF:2 * NF], preferred_element_type=f32)


def _precompute(x, u, b2d, W_e1, b_e1, W_n1, b_n1):
    return pl.pallas_call(
        _pre_body,
        out_shape=[jax.ShapeDtypeStruct((N, GW), f32),
                   jax.ShapeDtypeStruct((N, UF), f32)],
    )(x, u, b2d, W_e1, b_e1.reshape(1, 32), W_n1[0:NF],
      b_n1.reshape(1, 64))


_MESH = plsc.VectorSubcoreMesh(core_axis_name="c", subcore_axis_name="s",
                               num_cores=NC, num_subcores=NS)


@functools.partial(
    pl.kernel,
    out_type=[jax.ShapeDtypeStruct((E, GW), f32),
              jax.ShapeDtypeStruct((E, UF), f32)],
    mesh=_MESH,
    scratch_types=[
        pltpu.VMEM((KI, 128), jnp.int32),
        pltpu.VMEM((KI, 128), jnp.int32),
        pltpu.VMEM((CH, GW), f32),
        pltpu.VMEM((CH, UF), f32),
        pltpu.SemaphoreType.DMA,
    ],
)
def _gather_k(trow_hbm, tcol_hbm, row_hbm, col_hbm, g1_hbm, g2_hbm,
              idxr_v, idxc_v, r1_v, r2_v, sem):
    wid = lax.axis_index("s") * NC + lax.axis_index("c")

    @pl.loop(wid, NCHUNK, step=NW)
    def _(c):
        e0 = c * CH
        i0 = c * KI
        cpa = pltpu.async_copy(row_hbm.at[pl.ds(i0, KI)], idxr_v, sem)
        cpb = pltpu.async_copy(col_hbm.at[pl.ds(i0, KI)], idxc_v, sem)
        cpa.wait()
        cpb.wait()
        cps = []
        for j in range(KI):
            cps.append(pltpu.async_copy(
                trow_hbm.at[idxr_v.at[j]], r1_v.at[pl.ds(j * 128, 128)], sem))
            cps.append(pltpu.async_copy(
                tcol_hbm.at[idxc_v.at[j]], r2_v.at[pl.ds(j * 128, 128)], sem))
        for cp in cps:
            cp.wait()
        cp1 = pltpu.async_copy(r1_v, g1_hbm.at[pl.ds(e0, CH)], sem)
        cp2 = pltpu.async_copy(r2_v, g2_hbm.at[pl.ds(e0, CH)], sem)
        cp1.wait()
        cp2.wait()


def _edge_body(g1_ref, g2_ref, ea_ref, We1e_ref, We2_ref, be2_ref, We3_ref,
               be3_ref, Wn1e_ref, Wn2e_ref, bn2e_ref,
               eout_ref, h2_ref, eagg_ref):
    g1 = g1_ref[...]
    ea = ea_ref[...]
    h = jnp.maximum(
        g1[:, 0:32] + g2_ref[...]
        + jnp.dot(ea, We1e_ref[...], preferred_element_type=f32), 0.0)
    h = jnp.maximum(
        jnp.dot(h, We2_ref[...], preferred_element_type=f32) + be2_ref[...],
        0.0)
    e_new = jnp.dot(h, We3_ref[...], preferred_element_type=f32) + be3_ref[...]
    eout_ref[...] = e_new + ea
    h1 = jnp.maximum(
        g1[:, 32:96] + jnp.dot(e_new, Wn1e_ref[...],
                               preferred_element_type=f32), 0.0)
    h2_ref[...] = (jnp.dot(h1, Wn2e_ref[...], preferred_element_type=f32)
                   + bn2e_ref[...])
    oh = (g1[:, 96:97] == lax.broadcasted_iota(f32, (EB, B), 1)).astype(f32)
    ee = jnp.concatenate(
        [e_new, jnp.ones((EB, 1), f32), jnp.zeros((EB, UF - EF - 1), f32)],
        axis=1)
    upd = lax.dot_general(oh, ee, (((0,), (0,)), ((), ())),
                          preferred_element_type=f32)

    @pl.when(pl.program_id(0) == 0)
    def _():
        eagg_ref[...] = upd

    @pl.when(pl.program_id(0) > 0)
    def _():
        eagg_ref[...] += upd


def _edge_mlp(g1, g2, edge_attr, We1e, W_e2, b_e2, W_e3, b_e3, Wn1e, Wn2e,
              bn2e):
    full = lambda i: (0, 0)
    return pl.pallas_call(
        _edge_body,
        grid=(E // EB,),
        in_specs=[
            pl.BlockSpec((EB, GW), lambda i: (i, 0)),
            pl.BlockSpec((EB, UF), lambda i: (i, 0)),
            pl.BlockSpec((EB, EF), lambda i: (i, 0)),
            pl.BlockSpec((EF, 32), full),
            pl.BlockSpec((32, 32), full),
            pl.BlockSpec((1, 32), full),
            pl.BlockSpec((32, EF), full),
            pl.BlockSpec((1, EF), full),
            pl.BlockSpec((EF, 64), full),
            pl.BlockSpec((64, HW), full),
            pl.BlockSpec((1, HW), full),
        ],
        out_specs=[
            pl.BlockSpec((EB, EF), lambda i: (i, 0)),
            pl.BlockSpec((EB, HW), lambda i: (i, 0)),
            pl.BlockSpec((B, UF), full),
        ],
        out_shape=[jax.ShapeDtypeStruct((E, EF), f32),
                   jax.ShapeDtypeStruct((E, HW), f32),
                   jax.ShapeDtypeStruct((B, UF), f32)],
    )(g1, g2, edge_attr, We1e, W_e2, b_e2, W_e3, b_e3, Wn1e, Wn2e, bn2e)


@functools.partial(
    pl.kernel,
    out_type=jax.ShapeDtypeStruct((NC * NPAD, HW), f32),
    mesh=_MESH,
    scratch_types=[
        pltpu.VMEM((KI, 128), jnp.int32),
        pltpu.VMEM((CH, HW), f32),
        pltpu.VMEM_SHARED((NPAD, HW), f32),
        pltpu.SemaphoreType.DMA,
    ],
)
def _scatter_k(h2_hbm, col_hbm, zero_hbm, out_hbm, idx_v, buf_v, acc_sh, sem):
    cid = lax.axis_index("c")
    sid = lax.axis_index("s")
    wid = sid * NC + cid
    r0 = sid * RPS
    pltpu.sync_copy(zero_hbm.at[pl.ds(r0, RPS)], acc_sh.at[pl.ds(r0, RPS)])
    plsc.subcore_barrier()

    @pl.loop(wid, NCHUNK, step=NW)
    def _(c):
        e0 = c * CH
        cpa = pltpu.async_copy(col_hbm.at[pl.ds(c * KI, KI)], idx_v, sem)
        cpb = pltpu.async_copy(h2_hbm.at[pl.ds(e0, CH)], buf_v, sem)
        cpa.wait()
        cpb.wait()
        for j in range(KI):
            pltpu.sync_copy(buf_v.at[pl.ds(j * 128, 128)],
                            acc_sh.at[idx_v.at[j]], add=True)

    plsc.subcore_barrier()
    pltpu.sync_copy(acc_sh.at[pl.ds(r0, RPS)],
                    out_hbm.at[pl.ds(cid * NPAD + r0, RPS)])


def _node_body(x_ref, part_ref, u_ref, b_ref, eagg_ref, Wn3_ref, bn3_ref,
               Wn4_ref, bn4_ref, Wg1_ref, bg1_ref, Wg2_ref, bg2_ref,
               xout_ref, uout_ref):
    x = x_ref[...]
    u = u_ref[...]
    p = part_ref[0:N, :] + part_ref[NPAD:NPAD + N, :]
    agg = p[:, 0:64] / jnp.maximum(p[:, 64:65], 1.0)
    onehot = (b_ref[...] == lax.broadcasted_iota(jnp.int32, (N, B), 1)
              ).astype(f32)
    Wn3 = Wn3_ref[...]
    uW = jnp.dot(u, Wn3[NF + 64:NF + 64 + UF], preferred_element_type=f32)
    h = jnp.maximum(
        jnp.dot(x, Wn3[0:NF], preferred_element_type=f32)
        + jnp.dot(agg, Wn3[NF:NF + 64], preferred_element_type=f32)
        + jnp.dot(onehot, uW, preferred_element_type=f32) + bn3_ref[...], 0.0)
    x_new = jnp.dot(h, Wn4_ref[...], preferred_element_type=f32) + bn4_ref[...]
    xout_ref[...] = x_new + x
    xs = jnp.concatenate(
        [x_new, jnp.ones((N, 1), f32), jnp.zeros((N, 15), f32)], axis=1)
    nst = lax.dot_general(onehot, xs, (((0,), (0,)), ((), ())),
                          preferred_element_type=f32)
    n_agg = nst[:, 0:NF] / jnp.maximum(nst[:, NF:NF + 1], 1.0)
    eaggs = eagg_ref[...]
    e_agg = eaggs[:, 0:EF] / jnp.maximum(eaggs[:, EF:EF + 1], 1.0)
    g_in = jnp.concatenate([u, n_agg, e_agg], axis=1)
    hg = jnp.maximum(
        jnp.dot(g_in, Wg1_ref[...], preferred_element_type=f32)
        + bg1_ref[...], 0.0)
    uout_ref[...] = (jnp.dot(hg, Wg2_ref[...], preferred_element_type=f32)
                     + bg2_ref[...] + u)


def _node_global(x, parts, u, b2d, eaggp, W_n3, b_n3, W_n4, b_n4, W_g1, b_g1,
                 W_g2, b_g2):
    return pl.pallas_call(
        _node_body,
        out_shape=[jax.ShapeDtypeStruct((N, NF), f32),
                   jax.ShapeDtypeStruct((B, UF), f32)],
    )(x, parts, u, b2d, eaggp, W_n3, b_n3.reshape(1, 64), W_n4,
      b_n4.reshape(1, NF), W_g1, b_g1.reshape(1, 32), W_g2,
      b_g2.reshape(1, UF))


@jax.jit
def kernel(x, edge_index, edge_attr, u, batch,
           W_e1, b_e1, W_e2, b_e2, W_e3, b_e3,
           W_n1, b_n1, W_n2, b_n2, W_n3, b_n3, W_n4, b_n4,
           W_g1, b_g1, W_g2, b_g2):
    row2d = edge_index[0].reshape(NCHUNK * KI, 128)
    col2d = edge_index[1].reshape(NCHUNK * KI, 128)
    b2d = batch.reshape(N, 1)

    trow, tcol = _precompute(x, u, b2d, W_e1, b_e1, W_n1, b_n1)
    g1, g2 = _gather_k(trow, tcol, row2d, col2d)

    We1e = W_e1[2 * NF:2 * NF + EF]
    Wn1e = W_n1[NF:NF + EF]
    Wn2e = jnp.concatenate([W_n2, jnp.zeros((64, HW - 64), f32)], axis=1)
    bn2e = jnp.concatenate(
        [b_n2, jnp.ones((1,), f32), jnp.zeros((HW - 65,), f32)]
    ).reshape(1, HW)
    eout, h2, eaggp = _edge_mlp(g1, g2, edge_attr, We1e, W_e2,
                                b_e2.reshape(1, 32), W_e3,
                                b_e3.reshape(1, EF), Wn1e, Wn2e, bn2e)

    zeros = jnp.zeros((NPAD, HW), f32)
    parts = _scatter_k(h2, col2d, zeros)

    xout, uout = _node_global(x, parts, u, b2d, eaggp, W_n3, b_n3, W_n4,
                              b_n4, W_g1, b_g1, W_g2, b_g2)
    return xout, eout, uout


# speedup vs baseline: 5.6450x; 5.6450x over previous
import functools

import jax
import jax.numpy as jnp
from jax import lax
from jax.experimental import pallas as pl
from jax.experimental.pallas import tpu as pltpu
from jax.experimental.pallas import tpu_sc as plsc

N = 10000
E = 320000
B = 16
NF = 128
EF = 16
UF = 32

NC, NS = 2, 16
NW = NC * NS
NPAD = 10240
RPS = NPAD // NS
GW = 112
HW = 80
CH = 512
KI = CH // 128
NCHUNK = E // CH
EB = 2000
f32 = jnp.float32


def _pre_body(x_ref, u_ref, b_ref, We1_ref, be1_ref, Wn1x_ref, bn1_ref,
              trow_ref, tcol_ref):
    x = x_ref[...]
    We1 = We1_ref[...]
    onehot = (b_ref[...] == lax.broadcasted_iota(jnp.int32, (N, B), 1)
              ).astype(f32)
    uW = jnp.dot(u_ref[...], We1[2 * NF + EF:2 * NF + EF + UF, :],
                 preferred_element_type=f32)
    ta = (jnp.dot(x, We1[0:NF], preferred_element_type=f32)
          + jnp.dot(onehot, uW, preferred_element_type=f32) + be1_ref[...])
    tb = jnp.dot(x, Wn1x_ref[...], preferred_element_type=f32) + bn1_ref[...]
    bf = b_ref[...].astype(f32)
    trow_ref[...] = jnp.concatenate(
        [ta, tb, bf, jnp.zeros((N, GW - 97), f32)], axis=1)
    tcol_ref[...] = jnp.dot(x, We1[NF:2 * NF], preferred_element_type=f32)


def _precompute(x, u, b2d, W_e1, b_e1, W_n1, b_n1):
    return pl.pallas_call(
        _pre_body,
        out_shape=[jax.ShapeDtypeStruct((N, GW), f32),
                   jax.ShapeDtypeStruct((N, UF), f32)],
    )(x, u, b2d, W_e1, b_e1.reshape(1, 32), W_n1[0:NF],
      b_n1.reshape(1, 64))


@functools.lru_cache(maxsize=None)
def _sc_mesh():
    return plsc.VectorSubcoreMesh(core_axis_name="c", subcore_axis_name="s",
                                  num_cores=NC, num_subcores=NS)


@functools.lru_cache(maxsize=None)
def _build_gather():
    @functools.partial(
        pl.kernel,
        out_type=[jax.ShapeDtypeStruct((E, GW), f32),
                  jax.ShapeDtypeStruct((E, UF), f32)],
        mesh=_sc_mesh(),
        compiler_params=pltpu.CompilerParams(use_tc_tiling_on_sc=False),
        scratch_types=[
            pltpu.VMEM((KI, 128), jnp.int32),
            pltpu.VMEM((KI, 128), jnp.int32),
            pltpu.VMEM((CH, GW), f32),
            pltpu.VMEM((CH, UF), f32),
            pltpu.SemaphoreType.DMA,
        ],
    )
    def gather_body(trow_hbm, tcol_hbm, row_hbm, col_hbm, g1_hbm, g2_hbm,
                    idxr_v, idxc_v, r1_v, r2_v, sem):
        wid = lax.axis_index("s") * NC + lax.axis_index("c")

        @pl.loop(wid, NCHUNK, step=NW)
        def _(c):
            e0 = c * CH
            i0 = c * KI
            cpa = pltpu.async_copy(row_hbm.at[pl.ds(i0, KI)], idxr_v, sem)
            cpb = pltpu.async_copy(col_hbm.at[pl.ds(i0, KI)], idxc_v, sem)
            cpa.wait()
            cpb.wait()
            cps = []
            for j in range(KI):
                cps.append(pltpu.async_copy(
                    trow_hbm.at[idxr_v.at[j]],
                    r1_v.at[pl.ds(j * 128, 128)], sem))
                cps.append(pltpu.async_copy(
                    tcol_hbm.at[idxc_v.at[j]],
                    r2_v.at[pl.ds(j * 128, 128)], sem))
            for cp in cps:
                cp.wait()
            cp1 = pltpu.async_copy(r1_v, g1_hbm.at[pl.ds(e0, CH)], sem)
            cp2 = pltpu.async_copy(r2_v, g2_hbm.at[pl.ds(e0, CH)], sem)
            cp1.wait()
            cp2.wait()

    return gather_body


def _gather_k(trow, tcol, row2d, col2d):
    return _build_gather()(trow, tcol, row2d, col2d)


def _edge_body(g1_ref, g2_ref, ea_ref, We1e_ref, We2_ref, be2_ref, We3_ref,
               be3_ref, Wn1e_ref, Wn2e_ref, bn2e_ref,
               eout_ref, h2_ref, eagg_ref):
    g1 = g1_ref[...]
    ea = ea_ref[...]
    h = jnp.maximum(
        g1[:, 0:32] + g2_ref[...]
        + jnp.dot(ea, We1e_ref[...], preferred_element_type=f32), 0.0)
    h = jnp.maximum(
        jnp.dot(h, We2_ref[...], preferred_element_type=f32) + be2_ref[...],
        0.0)
    e_new = jnp.dot(h, We3_ref[...], preferred_element_type=f32) + be3_ref[...]
    eout_ref[...] = e_new + ea
    h1 = jnp.maximum(
        g1[:, 32:96] + jnp.dot(e_new, Wn1e_ref[...],
                               preferred_element_type=f32), 0.0)
    h2_ref[...] = (jnp.dot(h1, Wn2e_ref[...], preferred_element_type=f32)
                   + bn2e_ref[...])
    oh = (g1[:, 96:97].astype(jnp.int32)
          == lax.broadcasted_iota(jnp.int32, (EB, B), 1)).astype(f32)
    ee = jnp.concatenate(
        [e_new, jnp.ones((EB, 1), f32), jnp.zeros((EB, UF - EF - 1), f32)],
        axis=1)
    upd = lax.dot_general(oh, ee, (((0,), (0,)), ((), ())),
                          preferred_element_type=f32)

    @pl.when(pl.program_id(0) == 0)
    def _():
        eagg_ref[...] = upd

    @pl.when(pl.program_id(0) > 0)
    def _():
        eagg_ref[...] += upd


def _edge_mlp(g1, g2, edge_attr, We1e, W_e2, b_e2, W_e3, b_e3, Wn1e, Wn2e,
              bn2e):
    full = lambda i: (0, 0)
    return pl.pallas_call(
        _edge_body,
        grid=(E // EB,),
        in_specs=[
            pl.BlockSpec((EB, GW), lambda i: (i, 0)),
            pl.BlockSpec((EB, UF), lambda i: (i, 0)),
            pl.BlockSpec((EB, EF), lambda i: (i, 0)),
            pl.BlockSpec((EF, 32), full),
            pl.BlockSpec((32, 32), full),
            pl.BlockSpec((1, 32), full),
            pl.BlockSpec((32, EF), full),
            pl.BlockSpec((1, EF), full),
            pl.BlockSpec((EF, 64), full),
            pl.BlockSpec((64, HW), full),
            pl.BlockSpec((1, HW), full),
        ],
        out_specs=[
            pl.BlockSpec((EB, EF), lambda i: (i, 0)),
            pl.BlockSpec((EB, HW), lambda i: (i, 0)),
            pl.BlockSpec((B, UF), full),
        ],
        out_shape=[jax.ShapeDtypeStruct((E, EF), f32),
                   jax.ShapeDtypeStruct((E, HW), f32),
                   jax.ShapeDtypeStruct((B, UF), f32)],
    )(g1, g2, edge_attr, We1e, W_e2, b_e2, W_e3, b_e3, Wn1e, Wn2e, bn2e)


@functools.lru_cache(maxsize=None)
def _build_scatter():
    @functools.partial(
        pl.kernel,
        out_type=jax.ShapeDtypeStruct((NC * NPAD, HW), f32),
        mesh=_sc_mesh(),
        compiler_params=pltpu.CompilerParams(use_tc_tiling_on_sc=False),
        scratch_types=[
            pltpu.VMEM((KI, 128), jnp.int32),
            pltpu.VMEM((CH, HW), f32),
            pltpu.VMEM_SHARED((NPAD, HW), f32),
            pltpu.SemaphoreType.DMA,
        ],
    )
    def scatter_body(h2_hbm, col_hbm, zero_hbm, out_hbm, idx_v, buf_v,
                     acc_sh, sem):
        cid = lax.axis_index("c")
        sid = lax.axis_index("s")
        wid = sid * NC + cid
        r0 = sid * RPS
        pltpu.sync_copy(zero_hbm.at[pl.ds(r0, RPS)],
                        acc_sh.at[pl.ds(r0, RPS)])
        plsc.subcore_barrier()

        @pl.loop(wid, NCHUNK, step=NW)
        def _(c):
            e0 = c * CH
            cpa = pltpu.async_copy(col_hbm.at[pl.ds(c * KI, KI)], idx_v, sem)
            cpb = pltpu.async_copy(h2_hbm.at[pl.ds(e0, CH)], buf_v, sem)
            cpa.wait()
            cpb.wait()
            for j in range(KI):
                pltpu.sync_copy(buf_v.at[pl.ds(j * 128, 128)],
                                acc_sh.at[idx_v.at[j]], add=True)

        plsc.subcore_barrier()
        pltpu.sync_copy(acc_sh.at[pl.ds(r0, RPS)],
                        out_hbm.at[pl.ds(cid * NPAD + r0, RPS)])

    return scatter_body


def _scatter_k(h2, col2d, zeros):
    return _build_scatter()(h2, col2d, zeros)


def _node_body(x_ref, part_ref, u_ref, b_ref, eagg_ref, Wn3_ref, bn3_ref,
               Wn4_ref, bn4_ref, Wg1_ref, bg1_ref, Wg2_ref, bg2_ref,
               xout_ref, uout_ref):
    x = x_ref[...]
    u = u_ref[...]
    p = part_ref[0:N, :] + part_ref[NPAD:NPAD + N, :]
    agg = p[:, 0:64] / jnp.maximum(p[:, 64:65], 1.0)
    onehot = (b_ref[...] == lax.broadcasted_iota(jnp.int32, (N, B), 1)
              ).astype(f32)
    Wn3 = Wn3_ref[...]
    uW = jnp.dot(u, Wn3[NF + 64:NF + 64 + UF], preferred_element_type=f32)
    h = jnp.maximum(
        jnp.dot(x, Wn3[0:NF], preferred_element_type=f32)
        + jnp.dot(agg, Wn3[NF:NF + 64], preferred_element_type=f32)
        + jnp.dot(onehot, uW, preferred_element_type=f32) + bn3_ref[...], 0.0)
    x_new = jnp.dot(h, Wn4_ref[...], preferred_element_type=f32) + bn4_ref[...]
    xout_ref[...] = x_new + x
    xs = jnp.concatenate(
        [x_new, jnp.ones((N, 1), f32), jnp.zeros((N, 15), f32)], axis=1)
    nst = lax.dot_general(onehot, xs, (((0,), (0,)), ((), ())),
                          preferred_element_type=f32)
    n_agg = nst[:, 0:NF] / jnp.maximum(nst[:, NF:NF + 1], 1.0)
    eaggs = eagg_ref[...]
    e_agg = eaggs[:, 0:EF] / jnp.maximum(eaggs[:, EF:EF + 1], 1.0)
    g_in = jnp.concatenate([u, n_agg, e_agg], axis=1)
    hg = jnp.maximum(
        jnp.dot(g_in, Wg1_ref[...], preferred_element_type=f32)
        + bg1_ref[...], 0.0)
    uout_ref[...] = (jnp.dot(hg, Wg2_ref[...], preferred_element_type=f32)
                     + bg2_ref[...] + u)


def _node_global(x, parts, u, b2d, eaggp, W_n3, b_n3, W_n4, b_n4, W_g1, b_g1,
                 W_g2, b_g2):
    return pl.pallas_call(
        _node_body,
        out_shape=[jax.ShapeDtypeStruct((N, NF), f32),
                   jax.ShapeDtypeStruct((B, UF), f32)],
    )(x, parts, u, b2d, eaggp, W_n3, b_n3.reshape(1, 64), W_n4,
      b_n4.reshape(1, NF), W_g1, b_g1.reshape(1, 32), W_g2,
      b_g2.reshape(1, UF))


@jax.jit
def kernel(x, edge_index, edge_attr, u, batch,
           W_e1, b_e1, W_e2, b_e2, W_e3, b_e3,
           W_n1, b_n1, W_n2, b_n2, W_n3, b_n3, W_n4, b_n4,
           W_g1, b_g1, W_g2, b_g2):
    row2d = edge_index[0].reshape(NCHUNK * KI, 128)
    col2d = edge_index[1].reshape(NCHUNK * KI, 128)
    b2d = batch.reshape(N, 1)

    trow, tcol = _precompute(x, u, b2d, W_e1, b_e1, W_n1, b_n1)
    g1, g2 = _gather_k(trow, tcol, row2d, col2d)

    We1e = W_e1[2 * NF:2 * NF + EF]
    Wn1e = W_n1[NF:NF + EF]
    Wn2e = jnp.concatenate([W_n2, jnp.zeros((64, HW - 64), f32)], axis=1)
    bn2e = jnp.concatenate(
        [b_n2, jnp.ones((1,), f32), jnp.zeros((HW - 65,), f32)]
    ).reshape(1, HW)
    eout, h2, eaggp = _edge_mlp(g1, g2, edge_attr, We1e, W_e2,
                                b_e2.reshape(1, 32), W_e3,
                                b_e3.reshape(1, EF), Wn1e, Wn2e, bn2e)

    zeros = jnp.zeros((NPAD, HW), f32)
    parts = _scatter_k(h2, col2d, zeros)

    xout, uout = _node_global(x, parts, u, b2d, eaggp, W_n3, b_n3, W_n4,
                              b_n4, W_g1, b_g1, W_g2, b_g2)
    return xout, eout, uout


# speedup vs baseline: 6.7439x; 1.1947x over previous
import functools

import jax
import jax.numpy as jnp
from jax import lax
from jax.experimental import pallas as pl
from jax.experimental.pallas import tpu as pltpu
from jax.experimental.pallas import tpu_sc as plsc

N = 10000
E = 320000
B = 16
NF = 128
EF = 16
UF = 32

NC, NS = 2, 16
NW = NC * NS
NPAD = 10240
RPS = NPAD // NS
GW = 128
HW = 80
CH = 512
KI = CH // 128
NCHUNK = E // CH
EB = 2000
f32 = jnp.float32


def _pre_body(x_ref, u_ref, b_ref, We1_ref, be1_ref, Wn1x_ref, bn1_ref,
              trow_ref, tcol_ref):
    x = x_ref[...]
    We1 = We1_ref[...]
    onehot = (b_ref[...] == lax.broadcasted_iota(jnp.int32, (N, B), 1)
              ).astype(f32)
    uW = jnp.dot(u_ref[...], We1[2 * NF + EF:2 * NF + EF + UF, :],
                 preferred_element_type=f32)
    ta = (jnp.dot(x, We1[0:NF], preferred_element_type=f32)
          + jnp.dot(onehot, uW, preferred_element_type=f32) + be1_ref[...])
    tb = jnp.dot(x, Wn1x_ref[...], preferred_element_type=f32) + bn1_ref[...]
    bf = b_ref[...].astype(f32)
    trow_ref[...] = jnp.concatenate(
        [ta, tb, bf, jnp.zeros((N, GW - 97), f32)], axis=1)
    tcol_ref[...] = jnp.dot(x, We1[NF:2 * NF], preferred_element_type=f32)


def _precompute(x, u, b2d, W_e1, b_e1, W_n1, b_n1):
    return pl.pallas_call(
        _pre_body,
        out_shape=[jax.ShapeDtypeStruct((N, GW), f32),
                   jax.ShapeDtypeStruct((N, UF), f32)],
    )(x, u, b2d, W_e1, b_e1.reshape(1, 32), W_n1[0:NF],
      b_n1.reshape(1, 64))


@functools.lru_cache(maxsize=None)
def _sc_mesh():
    return plsc.VectorSubcoreMesh(core_axis_name="c", subcore_axis_name="s",
                                  num_cores=NC, num_subcores=NS)


@functools.lru_cache(maxsize=None)
def _build_gather1():
    @functools.partial(
        pl.kernel,
        out_type=jax.ShapeDtypeStruct((E, GW), f32),
        mesh=_sc_mesh(),
        scratch_types=[
            pltpu.VMEM((KI, 128), jnp.int32),
            pltpu.VMEM((CH, GW), f32),
            pltpu.SemaphoreType.DMA,
        ],
    )
    def gather1_body(trow_hbm, row_hbm, g1_hbm, idxr_v, r1_v, sem):
        wid = lax.axis_index("s") * NC + lax.axis_index("c")

        @pl.loop(wid, NCHUNK, step=NW)
        def _(c):
            e0 = c * CH
            i0 = c * KI
            pltpu.async_copy(row_hbm.at[pl.ds(i0, KI)], idxr_v, sem).wait()
            cps = [pltpu.async_copy(
                trow_hbm.at[idxr_v.at[j]],
                r1_v.at[pl.ds(j * 128, 128)], sem) for j in range(KI)]
            for cp in cps:
                cp.wait()
            pltpu.async_copy(r1_v, g1_hbm.at[pl.ds(e0, CH)], sem).wait()

    return gather1_body


@functools.lru_cache(maxsize=None)
def _build_gather2():
    @functools.partial(
        pl.kernel,
        out_type=jax.ShapeDtypeStruct((E, UF), f32),
        mesh=_sc_mesh(),
        compiler_params=pltpu.CompilerParams(use_tc_tiling_on_sc=False),
        scratch_types=[
            pltpu.VMEM((KI, 128), jnp.int32),
            pltpu.VMEM((CH, UF), f32),
            pltpu.SemaphoreType.DMA,
        ],
    )
    def gather2_body(tcol_hbm, col_hbm, g2_hbm, idxc_v, r2_v, sem):
        wid = lax.axis_index("s") * NC + lax.axis_index("c")

        @pl.loop(wid, NCHUNK, step=NW)
        def _(c):
            e0 = c * CH
            i0 = c * KI
            pltpu.async_copy(col_hbm.at[pl.ds(i0, KI)], idxc_v, sem).wait()
            cps = [pltpu.async_copy(
                tcol_hbm.at[idxc_v.at[j]],
                r2_v.at[pl.ds(j * 128, 128)], sem) for j in range(KI)]
            for cp in cps:
                cp.wait()
            pltpu.async_copy(r2_v, g2_hbm.at[pl.ds(e0, CH)], sem).wait()

    return gather2_body


def _gather_k(trow, tcol, row2d, col2d):
    return _build_gather1()(trow, row2d), _build_gather2()(tcol, col2d)


def _edge_body(g1_ref, g2_ref, ea_ref, We1e_ref, We2_ref, be2_ref, We3_ref,
               be3_ref, Wn1e_ref, Wn2e_ref, bn2e_ref,
               eout_ref, h2_ref, eagg_ref):
    g1 = g1_ref[...]
    ea = ea_ref[...]
    h = jnp.maximum(
        g1[:, 0:32] + g2_ref[...]
        + jnp.dot(ea, We1e_ref[...], preferred_element_type=f32), 0.0)
    h = jnp.maximum(
        jnp.dot(h, We2_ref[...], preferred_element_type=f32) + be2_ref[...],
        0.0)
    e_new = jnp.dot(h, We3_ref[...], preferred_element_type=f32) + be3_ref[...]
    eout_ref[...] = e_new + ea
    h1 = jnp.maximum(
        g1[:, 32:96] + jnp.dot(e_new, Wn1e_ref[...],
                               preferred_element_type=f32), 0.0)
    h2_ref[...] = (jnp.dot(h1, Wn2e_ref[...], preferred_element_type=f32)
                   + bn2e_ref[...])
    oh = (g1[:, 96:97].astype(jnp.int32)
          == lax.broadcasted_iota(jnp.int32, (EB, B), 1)).astype(f32)
    ee = jnp.concatenate(
        [e_new, jnp.ones((EB, 1), f32), jnp.zeros((EB, UF - EF - 1), f32)],
        axis=1)
    upd = lax.dot_general(oh, ee, (((0,), (0,)), ((), ())),
                          preferred_element_type=f32)

    @pl.when(pl.program_id(0) == 0)
    def _():
        eagg_ref[...] = upd

    @pl.when(pl.program_id(0) > 0)
    def _():
        eagg_ref[...] += upd


def _edge_mlp(g1, g2, edge_attr, We1e, W_e2, b_e2, W_e3, b_e3, Wn1e, Wn2e,
              bn2e):
    full = lambda i: (0, 0)
    return pl.pallas_call(
        _edge_body,
        grid=(E // EB,),
        in_specs=[
            pl.BlockSpec((EB, GW), lambda i: (i, 0)),
            pl.BlockSpec((EB, UF), lambda i: (i, 0)),
            pl.BlockSpec((EB, EF), lambda i: (i, 0)),
            pl.BlockSpec((EF, 32), full),
            pl.BlockSpec((32, 32), full),
            pl.BlockSpec((1, 32), full),
            pl.BlockSpec((32, EF), full),
            pl.BlockSpec((1, EF), full),
            pl.BlockSpec((EF, 64), full),
            pl.BlockSpec((64, HW), full),
            pl.BlockSpec((1, HW), full),
        ],
        out_specs=[
            pl.BlockSpec((EB, EF), lambda i: (i, 0)),
            pl.BlockSpec((EB, HW), lambda i: (i, 0)),
            pl.BlockSpec((B, UF), full),
        ],
        out_shape=[jax.ShapeDtypeStruct((E, EF), f32),
                   jax.ShapeDtypeStruct((E, HW), f32),
                   jax.ShapeDtypeStruct((B, UF), f32)],
    )(g1, g2, edge_attr, We1e, W_e2, b_e2, W_e3, b_e3, Wn1e, Wn2e, bn2e)


@functools.lru_cache(maxsize=None)
def _build_scatter():
    @functools.partial(
        pl.kernel,
        out_type=jax.ShapeDtypeStruct((NC * NPAD, HW), f32),
        mesh=_sc_mesh(),
        compiler_params=pltpu.CompilerParams(use_tc_tiling_on_sc=False),
        scratch_types=[
            pltpu.VMEM((KI, 128), jnp.int32),
            pltpu.VMEM((CH, HW), f32),
            pltpu.VMEM_SHARED((NPAD, HW), f32),
            pltpu.SemaphoreType.DMA,
        ],
    )
    def scatter_body(h2_hbm, col_hbm, zero_hbm, out_hbm, idx_v, buf_v,
                     acc_sh, sem):
        cid = lax.axis_index("c")
        sid = lax.axis_index("s")
        wid = sid * NC + cid
        r0 = sid * RPS
        pltpu.sync_copy(zero_hbm.at[pl.ds(r0, RPS)],
                        acc_sh.at[pl.ds(r0, RPS)])
        plsc.subcore_barrier()

        @pl.loop(wid, NCHUNK, step=NW)
        def _(c):
            e0 = c * CH
            cpa = pltpu.async_copy(col_hbm.at[pl.ds(c * KI, KI)], idx_v, sem)
            cpb = pltpu.async_copy(h2_hbm.at[pl.ds(e0, CH)], buf_v, sem)
            cpa.wait()
            cpb.wait()
            for j in range(KI):
                pltpu.sync_copy(buf_v.at[pl.ds(j * 128, 128)],
                                acc_sh.at[idx_v.at[j]], add=True)

        plsc.subcore_barrier()
        pltpu.sync_copy(acc_sh.at[pl.ds(r0, RPS)],
                        out_hbm.at[pl.ds(cid * NPAD + r0, RPS)])

    return scatter_body


def _scatter_k(h2, col2d, zeros):
    return _build_scatter()(h2, col2d, zeros)


def _node_body(x_ref, part_ref, u_ref, b_ref, eagg_ref, Wn3_ref, bn3_ref,
               Wn4_ref, bn4_ref, Wg1_ref, bg1_ref, Wg2_ref, bg2_ref,
               xout_ref, uout_ref):
    x = x_ref[...]
    u = u_ref[...]
    p = part_ref[0:N, :] + part_ref[NPAD:NPAD + N, :]
    agg = p[:, 0:64] / jnp.maximum(p[:, 64:65], 1.0)
    onehot = (b_ref[...] == lax.broadcasted_iota(jnp.int32, (N, B), 1)
              ).astype(f32)
    Wn3 = Wn3_ref[...]
    uW = jnp.dot(u, Wn3[NF + 64:NF + 64 + UF], preferred_element_type=f32)
    h = jnp.maximum(
        jnp.dot(x, Wn3[0:NF], preferred_element_type=f32)
        + jnp.dot(agg, Wn3[NF:NF + 64], preferred_element_type=f32)
        + jnp.dot(onehot, uW, preferred_element_type=f32) + bn3_ref[...], 0.0)
    x_new = jnp.dot(h, Wn4_ref[...], preferred_element_type=f32) + bn4_ref[...]
    xout_ref[...] = x_new + x
    xs = jnp.concatenate(
        [x_new, jnp.ones((N, 1), f32), jnp.zeros((N, 15), f32)], axis=1)
    nst = lax.dot_general(onehot, xs, (((0,), (0,)), ((), ())),
                          preferred_element_type=f32)
    n_agg = nst[:, 0:NF] / jnp.maximum(nst[:, NF:NF + 1], 1.0)
    eaggs = eagg_ref[...]
    e_agg = eaggs[:, 0:EF] / jnp.maximum(eaggs[:, EF:EF + 1], 1.0)
    g_in = jnp.concatenate([u, n_agg, e_agg], axis=1)
    hg = jnp.maximum(
        jnp.dot(g_in, Wg1_ref[...], preferred_element_type=f32)
        + bg1_ref[...], 0.0)
    uout_ref[...] = (jnp.dot(hg, Wg2_ref[...], preferred_element_type=f32)
                     + bg2_ref[...] + u)


def _node_global(x, parts, u, b2d, eaggp, W_n3, b_n3, W_n4, b_n4, W_g1, b_g1,
                 W_g2, b_g2):
    return pl.pallas_call(
        _node_body,
        out_shape=[jax.ShapeDtypeStruct((N, NF), f32),
                   jax.ShapeDtypeStruct((B, UF), f32)],
    )(x, parts, u, b2d, eaggp, W_n3, b_n3.reshape(1, 64), W_n4,
      b_n4.reshape(1, NF), W_g1, b_g1.reshape(1, 32), W_g2,
      b_g2.reshape(1, UF))


@jax.jit
def kernel(x, edge_index, edge_attr, u, batch,
           W_e1, b_e1, W_e2, b_e2, W_e3, b_e3,
           W_n1, b_n1, W_n2, b_n2, W_n3, b_n3, W_n4, b_n4,
           W_g1, b_g1, W_g2, b_g2):
    row2d = edge_index[0].reshape(NCHUNK * KI, 128)
    col2d = edge_index[1].reshape(NCHUNK * KI, 128)
    b2d = batch.reshape(N, 1)

    trow, tcol = _precompute(x, u, b2d, W_e1, b_e1, W_n1, b_n1)
    g1, g2 = _gather_k(trow, tcol, row2d, col2d)

    We1e = W_e1[2 * NF:2 * NF + EF]
    Wn1e = W_n1[NF:NF + EF]
    Wn2e = jnp.concatenate([W_n2, jnp.zeros((64, HW - 64), f32)], axis=1)
    bn2e = jnp.concatenate(
        [b_n2, jnp.ones((1,), f32), jnp.zeros((HW - 65,), f32)]
    ).reshape(1, HW)
    eout, h2, eaggp = _edge_mlp(g1, g2, edge_attr, We1e, W_e2,
                                b_e2.reshape(1, 32), W_e3,
                                b_e3.reshape(1, EF), Wn1e, Wn2e, bn2e)

    zeros = jnp.zeros((NPAD, HW), f32)
    parts = _scatter_k(h2, col2d, zeros)

    xout, uout = _node_global(x, parts, u, b2d, eaggp, W_n3, b_n3, W_n4,
                              b_n4, W_g1, b_g1, W_g2, b_g2)
    return xout, eout, uout
